# baseline (speedup 1.0000x reference)
import jax
import jax.numpy as jnp
from jax import lax
from jax.experimental import pallas as pl
from jax.experimental.pallas import tpu as pltpu
from jax.experimental.pallas import tpu_sc as plsc

N = 10000
D = 128
E = 160000

NS = 16
LANES = 16

CHUNK = 128
NCH = 80
EPT = NCH * CHUNK
E_PAD = EPT * NS
N_PAD = 10240
ROWS_PT = N_PAD // NS
ZROWS = 16


def _sc_body(x_hbm, s0_hbm, d0_hbm, s1_hbm, d1_hbm,
             agg0_hbm, deg0_hbm, agg1_hbm, deg1_hbm,
             srcb0, srcb1, srcb2, srcb3, dstb0, dstb1, dstb2, dstb3,
             rows0, rows1, zbuf, agg_s,
             isem0, isem1, isem2, isem3, gsem0, gsem1, ssem0, ssem1):
    c = lax.axis_index("c")
    s = lax.axis_index("s")
    f32 = jnp.float32

    srcb = (srcb0, srcb1, srcb2, srcb3)
    dstb = (dstb0, dstb1, dstb2, dstb3)
    rows = (rows0, rows1)
    isem = (isem0, isem1, isem2, isem3)
    gsem = (gsem0, gsem1)
    ssem = (ssem0, ssem1)

    base = s * ROWS_PT
    ebase = s * EPT

    def fill(buf, nrows, val):
        def frow(r, _):
            def fcol(k, _):
                buf[r, pl.ds(k * LANES, LANES)] = jnp.full((LANES,), val, f32)
                return 0
            return lax.fori_loop(0, D // LANES, fcol, 0)
        lax.fori_loop(0, nrows, frow, 0)

    def zero_table():
        def zs(k, _):
            pltpu.sync_copy(zbuf, agg_s.at[pl.ds(base + k * ZROWS, ZROWS)])
            return 0
        lax.fori_loop(0, ROWS_PT // ZROWS, zs, 0)

    def write_out(out_hbm):
        def wr(k, _):
            oblk = pl.ds(base + k * CHUNK, CHUNK)
            pltpu.sync_copy(agg_s.at[oblk], rows0)
            pltpu.sync_copy(rows0, out_hbm.at[oblk])
            return 0
        lax.fori_loop(0, ROWS_PT // CHUNK, wr, 0)

    fill(zbuf, ZROWS, 0.0)
    zero_table()
    plsc.subcore_barrier()

    def pass1(s_hbm, d_hbm):
        def issue_idx(j, ib):
            off = pl.ds(ebase + j * CHUNK, CHUNK)
            pltpu.async_copy(s_hbm.at[off], srcb[ib], isem[ib])
            pltpu.async_copy(d_hbm.at[off], dstb[ib], isem[ib])

        def wait_idx(j, ib):
            off = pl.ds(ebase + j * CHUNK, CHUNK)
            pltpu.make_async_copy(s_hbm.at[off], srcb[ib], isem[ib]).wait()
            pltpu.make_async_copy(d_hbm.at[off], dstb[ib], isem[ib]).wait()

        def drain_scatter(rb, ib):
            pltpu.make_async_copy(rows[rb], agg_s.at[dstb[ib]],
                                  ssem[rb]).wait()

        H = CHUNK // 2

        def issue_gather(rb, ib):
            pltpu.async_copy(x_hbm.at[srcb[ib].at[pl.ds(0, H)]],
                             rows[rb].at[pl.ds(0, H)], gsem[rb])
            pltpu.async_copy(x_hbm.at[srcb[ib].at[pl.ds(H, H)]],
                             rows[rb].at[pl.ds(H, H)], gsem[rb])

        def wait_gather(rb, ib):
            pltpu.make_async_copy(x_hbm.at[srcb[ib].at[pl.ds(0, H)]],
                                  rows[rb].at[pl.ds(0, H)], gsem[rb]).wait()
            pltpu.make_async_copy(x_hbm.at[srcb[ib].at[pl.ds(H, H)]],
                                  rows[rb].at[pl.ds(H, H)], gsem[rb]).wait()

        def issue_scatter(rb, ib):
            pltpu.async_copy(rows[rb], agg_s.at[dstb[ib]], ssem[rb], add=True)

        issue_idx(0, 0)
        issue_idx(1, 1)

        def body(i, _):
            j0 = 4 * i
            for t in range(4):
                j = j0 + t
                rb, ib = t % 2, t
                pb, pib = (t + 1) % 2, (t - 1) % 4

                def drain(rb=rb, ib=ib):
                    drain_scatter(rb, ib)

                if t < 2:
                    pl.when(i > 0)(drain)
                else:
                    drain()
                wait_idx(j, ib)
                issue_gather(rb, ib)

                def prefetch(j=j, ib=ib):
                    issue_idx(j + 2, (ib + 2) % 4)

                if t < 2:
                    prefetch()
                else:
                    pl.when(i < NCH // 4 - 1)(prefetch)

                def tail(j=j, pb=pb, pib=pib):
                    wait_gather(pb, pib)
                    issue_scatter(pb, pib)

                if t == 0:
                    pl.when(i > 0)(tail)
                else:
                    tail()
            return 0
        lax.fori_loop(0, NCH // 4, body, 0)

        wait_gather(1, 3)
        issue_scatter(1, 3)
        drain_scatter(0, 2)
        drain_scatter(1, 3)

    @pl.when(c == 0)
    def _():
        pass1(s0_hbm, d0_hbm)

    @pl.when(c == 1)
    def _():
        pass1(s1_hbm, d1_hbm)

    plsc.subcore_barrier()

    @pl.when(c == 0)
    def _():
        write_out(agg0_hbm)

    @pl.when(c == 1)
    def _():
        write_out(agg1_hbm)

    plsc.subcore_barrier()
    zero_table()
    fill(rows1, CHUNK, 1.0)
    plsc.subcore_barrier()

    def pass2(d_hbm):
        def load_didx(j, p):
            pltpu.sync_copy(d_hbm.at[pl.ds(ebase + j * CHUNK, CHUNK)],
                            dstb[p])

        def drain2(p):
            pltpu.make_async_copy(rows1, agg_s.at[dstb[p]], ssem[p]).wait()

        def issue2(p):
            pltpu.async_copy(rows1, agg_s.at[dstb[p]], ssem[p], add=True)

        load_didx(0, 0)
        issue2(0)

        def body(i, _):
            j0 = 2 * i
            @pl.when(i > 0)
            def _():
                drain2(1)
            load_didx(j0 + 1, 1)
            issue2(1)
            drain2(0)
            @pl.when(i < NCH // 2 - 1)
            def _():
                load_didx(j0 + 2, 0)
                issue2(0)
            return 0
        lax.fori_loop(0, NCH // 2, body, 0)
        drain2(1)

    @pl.when(c == 0)
    def _():
        pass2(d0_hbm)

    @pl.when(c == 1)
    def _():
        pass2(d1_hbm)

    plsc.subcore_barrier()

    @pl.when(c == 0)
    def _():
        write_out(deg0_hbm)

    @pl.when(c == 1)
    def _():
        write_out(deg1_hbm)


@jax.jit
def _sc_aggregate(x, s0, d0, s1, d1):
    f32 = jnp.float32
    run = pl.kernel(
        _sc_body,
        out_type=[
            jax.ShapeDtypeStruct((N_PAD, D), f32),
            jax.ShapeDtypeStruct((N_PAD, D), f32),
            jax.ShapeDtypeStruct((N_PAD, D), f32),
            jax.ShapeDtypeStruct((N_PAD, D), f32),
        ],
        mesh=plsc.VectorSubcoreMesh(core_axis_name="c", subcore_axis_name="s"),
        scratch_types=(
            [pltpu.VMEM((CHUNK,), jnp.int32)] * 4 +
            [pltpu.VMEM((CHUNK,), jnp.int32)] * 4 +
            [pltpu.VMEM((CHUNK, D), f32)] * 2 +
            [pltpu.VMEM((ZROWS, D), f32)] +
            [pltpu.VMEM_SHARED((N_PAD, D), f32)] +
            [pltpu.SemaphoreType.DMA] * 8
        ),
    )
    return run(x, s0, d0, s1, d1)


def _tc_body(x_ref, a0_ref, a1_ref, g0_ref, g1_ref, w0_ref, w1_ref, ws_ref,
             out_ref):
    inv0 = 1.0 / jnp.maximum(g0_ref[:, 0:1], 1.0)
    inv1 = 1.0 / jnp.maximum(g1_ref[:, 0:1], 1.0)
    acc = jnp.dot(a0_ref[...] * inv0, w0_ref[...],
                  preferred_element_type=jnp.float32)
    acc += jnp.dot(a1_ref[...] * inv1, w1_ref[...],
                   preferred_element_type=jnp.float32)
    acc += jnp.dot(x_ref[...], ws_ref[...], preferred_element_type=jnp.float32)
    out_ref[...] = jnp.maximum(acc, 0.0)


@jax.jit
def _tc_combine(x, agg0, agg1, deg0, deg1, W0, W1, Ws):
    blk = 2000
    grid = (N // blk,)
    row_spec = pl.BlockSpec((blk, D), lambda i: (i, 0))
    deg_spec = pl.BlockSpec((blk, D), lambda i: (i, 0))
    w_spec = pl.BlockSpec((D, D), lambda i: (0, 0))
    return pl.pallas_call(
        _tc_body,
        grid=grid,
        in_specs=[row_spec, row_spec, row_spec, deg_spec, deg_spec,
                  w_spec, w_spec, w_spec],
        out_specs=row_spec,
        out_shape=jax.ShapeDtypeStruct((N, D), jnp.float32),
    )(x, agg0, agg1, deg0, deg1, W0, W1, Ws)


def kernel(x, edge_index_rel0, edge_index_rel1, W_rel0, W_rel1, W_self):
    pad = E_PAD - E
    zpad = jnp.zeros((pad,), jnp.int32)
    npad = jnp.full((pad,), N, jnp.int32)

    s0 = jnp.concatenate([edge_index_rel0[0], zpad])
    d0 = jnp.concatenate([edge_index_rel0[1], npad])
    s1 = jnp.concatenate([edge_index_rel1[0], zpad])
    d1 = jnp.concatenate([edge_index_rel1[1], npad])

    agg0, dtab0, agg1, dtab1 = _sc_aggregate(x, s0, d0, s1, d1)
    return _tc_combine(x, agg0, agg1, dtab0, dtab1,
                       W_rel0, W_rel1, W_self)


# speedup vs baseline: 1.0151x; 1.0151x over previous
import jax
import jax.numpy as jnp
from jax import lax
from jax.experimental import pallas as pl
from jax.experimental.pallas import tpu as pltpu
from jax.experimental.pallas import tpu_sc as plsc

N = 10000
D = 128
E = 160000

NS = 16
LANES = 16

CHUNK = 128
NCH = 80
EPT = NCH * CHUNK
E_PAD = EPT * NS
N_PAD = 10240
ROWS_PT = N_PAD // NS
ZROWS = 16


def _sc_body(x_hbm, s0_hbm, d0_hbm, s1_hbm, d1_hbm,
             agg0_hbm, deg0_hbm, agg1_hbm, deg1_hbm,
             srcb0, srcb1, srcb2, srcb3, dstb0, dstb1, dstb2, dstb3,
             rows0, rows1, zbuf, agg_s,
             isem0, isem1, isem2, isem3, gsem0, gsem1, ssem0, ssem1):
    c = lax.axis_index("c")
    s = lax.axis_index("s")
    f32 = jnp.float32

    srcb = (srcb0, srcb1, srcb2, srcb3)
    dstb = (dstb0, dstb1, dstb2, dstb3)
    rows = (rows0, rows1)
    isem = (isem0, isem1, isem2, isem3)
    gsem = (gsem0, gsem1)
    ssem = (ssem0, ssem1)

    base = s * ROWS_PT
    ebase = s * EPT

    def fill(buf, nrows, val):
        def frow(r, _):
            def fcol(k, _):
                buf[r, pl.ds(k * LANES, LANES)] = jnp.full((LANES,), val, f32)
                return 0
            return lax.fori_loop(0, D // LANES, fcol, 0)
        lax.fori_loop(0, nrows, frow, 0)

    def zero_table():
        def zs(k, _):
            pltpu.async_copy(zbuf, agg_s.at[pl.ds(base + k * ZROWS, ZROWS)],
                             isem0)
            return 0
        lax.fori_loop(0, ROWS_PT // ZROWS, zs, 0)
        def zw(k, _):
            pltpu.make_async_copy(
                zbuf, agg_s.at[pl.ds(base + k * ZROWS, ZROWS)], isem0).wait()
            return 0
        lax.fori_loop(0, ROWS_PT // ZROWS, zw, 0)

    def write_out(out_hbm):
        nh = ROWS_PT // CHUNK
        for k in range(nh):
            rb = k % 2
            oblk = pl.ds(base + k * CHUNK, CHUNK)
            if k >= 2:
                pblk = pl.ds(base + (k - 2) * CHUNK, CHUNK)
                pltpu.make_async_copy(rows[rb], out_hbm.at[pblk],
                                      gsem[rb]).wait()
            pltpu.sync_copy(agg_s.at[oblk], rows[rb])
            pltpu.async_copy(rows[rb], out_hbm.at[oblk], gsem[rb])
        for k in range(nh - 2, nh):
            rb = k % 2
            oblk = pl.ds(base + k * CHUNK, CHUNK)
            pltpu.make_async_copy(rows[rb], out_hbm.at[oblk], gsem[rb]).wait()

    fill(zbuf, ZROWS, 0.0)
    zero_table()
    plsc.subcore_barrier()

    def pass1(s_hbm, d_hbm):
        def issue_idx(j, ib):
            off = pl.ds(ebase + j * CHUNK, CHUNK)
            pltpu.async_copy(s_hbm.at[off], srcb[ib], isem[ib])
            pltpu.async_copy(d_hbm.at[off], dstb[ib], isem[ib])

        def wait_idx(j, ib):
            off = pl.ds(ebase + j * CHUNK, CHUNK)
            pltpu.make_async_copy(s_hbm.at[off], srcb[ib], isem[ib]).wait()
            pltpu.make_async_copy(d_hbm.at[off], dstb[ib], isem[ib]).wait()

        def drain_scatter(rb, ib):
            pltpu.make_async_copy(rows[rb], agg_s.at[dstb[ib]],
                                  ssem[rb]).wait()

        H = CHUNK // 2

        def issue_gather(rb, ib):
            pltpu.async_copy(x_hbm.at[srcb[ib].at[pl.ds(0, H)]],
                             rows[rb].at[pl.ds(0, H)], gsem[rb])
            pltpu.async_copy(x_hbm.at[srcb[ib].at[pl.ds(H, H)]],
                             rows[rb].at[pl.ds(H, H)], gsem[rb])

        def wait_gather(rb, ib):
            pltpu.make_async_copy(x_hbm.at[srcb[ib].at[pl.ds(0, H)]],
                                  rows[rb].at[pl.ds(0, H)], gsem[rb]).wait()
            pltpu.make_async_copy(x_hbm.at[srcb[ib].at[pl.ds(H, H)]],
                                  rows[rb].at[pl.ds(H, H)], gsem[rb]).wait()

        def issue_scatter(rb, ib):
            pltpu.async_copy(rows[rb], agg_s.at[dstb[ib]], ssem[rb], add=True)

        issue_idx(0, 0)
        issue_idx(1, 1)

        def body(i, _):
            j0 = 4 * i
            for t in range(4):
                j = j0 + t
                rb, ib = t % 2, t
                pb, pib = (t + 1) % 2, (t - 1) % 4

                def drain(rb=rb, ib=ib):
                    drain_scatter(rb, ib)

                if t < 2:
                    pl.when(i > 0)(drain)
                else:
                    drain()
                wait_idx(j, ib)
                issue_gather(rb, ib)

                def prefetch(j=j, ib=ib):
                    issue_idx(j + 2, (ib + 2) % 4)

                if t < 2:
                    prefetch()
                else:
                    pl.when(i < NCH // 4 - 1)(prefetch)

                def tail(j=j, pb=pb, pib=pib):
                    wait_gather(pb, pib)
                    issue_scatter(pb, pib)

                if t == 0:
                    pl.when(i > 0)(tail)
                else:
                    tail()
            return 0
        lax.fori_loop(0, NCH // 4, body, 0)

        wait_gather(1, 3)
        issue_scatter(1, 3)
        drain_scatter(0, 2)
        drain_scatter(1, 3)

    @pl.when(c == 0)
    def _():
        pass1(s0_hbm, d0_hbm)

    @pl.when(c == 1)
    def _():
        pass1(s1_hbm, d1_hbm)

    plsc.subcore_barrier()

    @pl.when(c == 0)
    def _():
        write_out(agg0_hbm)

    @pl.when(c == 1)
    def _():
        write_out(agg1_hbm)

    plsc.subcore_barrier()
    zero_table()
    fill(rows1, CHUNK, 1.0)
    plsc.subcore_barrier()

    def pass2(d_hbm):
        def load_didx(j, p):
            pltpu.sync_copy(d_hbm.at[pl.ds(ebase + j * CHUNK, CHUNK)],
                            dstb[p])

        def drain2(p):
            pltpu.make_async_copy(rows1, agg_s.at[dstb[p]], ssem[p]).wait()

        def issue2(p):
            pltpu.async_copy(rows1, agg_s.at[dstb[p]], ssem[p], add=True)

        load_didx(0, 0)
        issue2(0)

        def body(i, _):
            j0 = 2 * i
            @pl.when(i > 0)
            def _():
                drain2(1)
            load_didx(j0 + 1, 1)
            issue2(1)
            drain2(0)
            @pl.when(i < NCH // 2 - 1)
            def _():
                load_didx(j0 + 2, 0)
                issue2(0)
            return 0
        lax.fori_loop(0, NCH // 2, body, 0)
        drain2(1)

    @pl.when(c == 0)
    def _():
        pass2(d0_hbm)

    @pl.when(c == 1)
    def _():
        pass2(d1_hbm)

    plsc.subcore_barrier()

    @pl.when(c == 0)
    def _():
        write_out(deg0_hbm)

    @pl.when(c == 1)
    def _():
        write_out(deg1_hbm)


@jax.jit
def _sc_aggregate(x, s0, d0, s1, d1):
    f32 = jnp.float32
    run = pl.kernel(
        _sc_body,
        out_type=[
            jax.ShapeDtypeStruct((N_PAD, D), f32),
            jax.ShapeDtypeStruct((N_PAD, D), f32),
            jax.ShapeDtypeStruct((N_PAD, D), f32),
            jax.ShapeDtypeStruct((N_PAD, D), f32),
        ],
        mesh=plsc.VectorSubcoreMesh(core_axis_name="c", subcore_axis_name="s"),
        scratch_types=(
            [pltpu.VMEM((CHUNK,), jnp.int32)] * 4 +
            [pltpu.VMEM((CHUNK,), jnp.int32)] * 4 +
            [pltpu.VMEM((CHUNK, D), f32)] * 2 +
            [pltpu.VMEM((ZROWS, D), f32)] +
            [pltpu.VMEM_SHARED((N_PAD, D), f32)] +
            [pltpu.SemaphoreType.DMA] * 8
        ),
    )
    return run(x, s0, d0, s1, d1)


def _tc_body(x_ref, a0_ref, a1_ref, g0_ref, g1_ref, w0_ref, w1_ref, ws_ref,
             out_ref):
    inv0 = 1.0 / jnp.maximum(g0_ref[:, 0:1], 1.0)
    inv1 = 1.0 / jnp.maximum(g1_ref[:, 0:1], 1.0)
    acc = jnp.dot(a0_ref[...] * inv0, w0_ref[...],
                  preferred_element_type=jnp.float32)
    acc += jnp.dot(a1_ref[...] * inv1, w1_ref[...],
                   preferred_element_type=jnp.float32)
    acc += jnp.dot(x_ref[...], ws_ref[...], preferred_element_type=jnp.float32)
    out_ref[...] = jnp.maximum(acc, 0.0)


@jax.jit
def _tc_combine(x, agg0, agg1, deg0, deg1, W0, W1, Ws):
    blk = 2000
    grid = (N // blk,)
    row_spec = pl.BlockSpec((blk, D), lambda i: (i, 0))
    deg_spec = pl.BlockSpec((blk, D), lambda i: (i, 0))
    w_spec = pl.BlockSpec((D, D), lambda i: (0, 0))
    return pl.pallas_call(
        _tc_body,
        grid=grid,
        in_specs=[row_spec, row_spec, row_spec, deg_spec, deg_spec,
                  w_spec, w_spec, w_spec],
        out_specs=row_spec,
        out_shape=jax.ShapeDtypeStruct((N, D), jnp.float32),
    )(x, agg0, agg1, deg0, deg1, W0, W1, Ws)


def kernel(x, edge_index_rel0, edge_index_rel1, W_rel0, W_rel1, W_self):
    pad = E_PAD - E
    zpad = jnp.zeros((pad,), jnp.int32)
    npad = jnp.full((pad,), N, jnp.int32)

    s0 = jnp.concatenate([edge_index_rel0[0], zpad])
    d0 = jnp.concatenate([edge_index_rel0[1], npad])
    s1 = jnp.concatenate([edge_index_rel1[0], zpad])
    d1 = jnp.concatenate([edge_index_rel1[1], npad])

    agg0, dtab0, agg1, dtab1 = _sc_aggregate(x, s0, d0, s1, d1)
    return _tc_combine(x, agg0, agg1, dtab0, dtab1,
                       W_rel0, W_rel1, W_self)
